# Initial kernel scaffold; baseline (speedup 1.0000x reference)
import jax
import jax.numpy as jnp
from jax.experimental import pallas as pl


def kernel(ptcloud, cubic_features):
    raise NotImplementedError("write your pallas kernel here")


# speedup vs baseline: 2.3748x; 2.3748x over previous
import functools

import jax
import jax.numpy as jnp
from jax.experimental import pallas as pl
from jax.experimental.pallas import tpu as pltpu
from jax.experimental.pallas import tpu_sc as plsc


def _corner_index_body(n_per_batch, grid_cells, zero_row, scale, dims, pts_ref, out_ref):
    blk = out_ref.shape[0]
    b = (pl.program_id(0) * blk) // n_per_batch
    boff = b * grid_cells
    pts = pts_ref[...]
    p = (pts + 1.0) * scale
    low = jnp.floor(p).astype(jnp.int32)
    lx = low[:, 0:1]
    ly = low[:, 1:2]
    lz = low[:, 2:3]
    sx, sy, sz = dims
    k = jax.lax.broadcasted_iota(jnp.int32, (1, 8), 1)
    offx = (k >> 2) & 1
    offy = (k >> 1) & 1
    offz = k & 1
    cx = lx + offx
    cy = ly + offy
    cz = lz + offz
    valid = (
        (cx >= 0) & (cx < sx)
        & (cy >= 0) & (cy < sy)
        & (cz >= 0) & (cz < sz)
    )
    flat = cx * (sy * sz) + cy * sz + cz + boff
    out_ref[...] = jnp.where(valid, flat, zero_row)


def _compute_indices(pts2d, n_per_batch, grid_cells, zero_row, scale, dims):
    total = pts2d.shape[0]
    blk = min(2048, total)
    body = functools.partial(
        _corner_index_body, n_per_batch, grid_cells, zero_row, scale, dims
    )
    return pl.pallas_call(
        body,
        grid=(total // blk,),
        in_specs=[pl.BlockSpec((blk, 3), lambda i: (i, 0))],
        out_specs=pl.BlockSpec((blk, 8), lambda i: (i, 0)),
        out_shape=jax.ShapeDtypeStruct((total, 8), jnp.int32),
    )(pts2d)


def _sc_gather(table, idx_row, num_idx, channels, window):
    mesh = plsc.VectorSubcoreMesh(core_axis_name="c", subcore_axis_name="s")

    @functools.partial(
        pl.kernel,
        out_type=jax.ShapeDtypeStruct((num_idx, channels), jnp.float32),
        mesh=mesh,
    )
    def gather_kernel(table_hbm, idx_hbm, out_hbm):
        def body(i_vmem, o_vmem):
            pltpu.sync_copy(table_hbm.at[i_vmem.at[0]], o_vmem)

        pltpu.emit_pipeline(
            body,
            grid=(num_idx // window,),
            in_specs=[pl.BlockSpec((1, window), lambda i: (0, i))],
            out_specs=[pl.BlockSpec((window, channels), lambda i: (i, 0))],
            core_axis_name=("c", "s"),
            dimension_semantics=(pltpu.PARALLEL,),
        )(idx_hbm, out_hbm)

    return gather_kernel(table, idx_row)


def kernel(ptcloud, cubic_features):
    B, C, sx, sy, sz = cubic_features.shape
    N = ptcloud.shape[1]
    S = sx * sy * sz
    zero_row = B * S
    scale = (sx - 1) * 0.5

    table = cubic_features.reshape(B, C, S).transpose(0, 2, 1).reshape(B * S, C)
    table = jnp.concatenate([table, jnp.zeros((8, C), table.dtype)], axis=0)

    pts2d = ptcloud.reshape(B * N, 3)
    idx = _compute_indices(pts2d, N, S, zero_row, scale, (sx, sy, sz))
    idx_row = idx.reshape(1, B * N * 8)
    out = _sc_gather(table, idx_row, B * N * 8, C, 128)
    return out.reshape(B, N, 8, C)
